# baseline (speedup 1.0000x reference)
import functools

import jax
import jax.numpy as jnp
from jax import lax
from jax.experimental import pallas as pl
from jax.experimental.pallas import tpu as pltpu
from jax.experimental.pallas import tpu_sc as plsc

NBINS = 15
PAD_BINS = 16

NSUB = 16
LANES = 16
IDX_CHUNK = 128


def _sc_gather_body(flat_ref, labels_ref, xl_out,
                    lab_v, idx_v, xl_v, sem, *, rows_per, ncols):
    tid = lax.axis_index("s")
    row0 = tid * rows_per
    nchunks = rows_per // LANES

    pltpu.sync_copy(labels_ref.at[pl.ds(row0, rows_per)], lab_v)

    chunks_per_idx_row = IDX_CHUNK // LANES

    def build_idx(c, carry):
        lab = lab_v[pl.ds(c * LANES, LANES)]
        rows = row0 + c * LANES + lax.iota(jnp.int32, LANES)
        idx = rows * ncols + lab
        j = c // chunks_per_idx_row
        off = (c % chunks_per_idx_row) * LANES
        idx_v[j, pl.ds(off, LANES)] = idx
        return carry

    lax.fori_loop(0, nchunks, build_idx, 0)

    copies = []
    for j in range(rows_per // IDX_CHUNK):
        copies.append(
            pltpu.async_copy(
                flat_ref.at[idx_v.at[j]],
                xl_v.at[pl.ds(j * IDX_CHUNK, IDX_CHUNK)],
                sem,
            )
        )
    for c in copies:
        c.wait()

    pltpu.sync_copy(xl_v, xl_out.at[pl.ds(row0, rows_per)])


def _soft_ece_kernel(logits_ref, xlab_ref, out_ref, acc_ref, *, nblocks):
    i = pl.program_id(0)

    @pl.when(i == 0)
    def _init():
        acc_ref[...] = jnp.zeros_like(acc_ref)

    x = logits_ref[...]
    b, c = x.shape
    m = jnp.max(x, axis=1, keepdims=True)
    s = jnp.sum(jnp.exp(x), axis=1, keepdims=True)
    x_lab = xlab_ref[0]

    m16 = jnp.broadcast_to(m, (b, PAD_BINS))
    s16 = jnp.broadcast_to(s, (b, PAD_BINS))
    xl16 = jnp.broadcast_to(x_lab, (b, PAD_BINS))
    inv_s = 1.0 / s16
    max_prob = jnp.exp(m16) * inv_s
    pred_prob = jnp.exp(xl16) * inv_s

    bin_width = jnp.float32(1.0 / NBINS)
    bins = jnp.floor(max_prob / bin_width).astype(jnp.int32)
    bins = jnp.clip(bins, 0, NBINS - 1)

    bin_iota = jax.lax.broadcasted_iota(jnp.int32, (b, PAD_BINS), 1)
    onehot = (bins == bin_iota).astype(jnp.float32)

    acc_ref[0:1, :] += jnp.sum(onehot, axis=0, keepdims=True)
    acc_ref[1:2, :] += jnp.sum(onehot * max_prob, axis=0, keepdims=True)
    acc_ref[2:3, :] += jnp.sum(onehot * pred_prob, axis=0, keepdims=True)

    @pl.when(i == nblocks - 1)
    def _finish():
        counts = acc_ref[0:1, :]
        conf_sum = acc_ref[1:2, :]
        acc_sum = acc_ref[2:3, :]
        safe = jnp.maximum(counts, 1.0)
        conf_mean = jnp.where(counts > 0, conf_sum / safe, 0.0)
        acc_mean = jnp.where(counts > 0, acc_sum / safe, 0.0)
        num = jnp.sum(counts * jnp.abs(conf_mean - acc_mean), keepdims=True)
        den = jnp.sum(counts, keepdims=True)
        out_ref[...] = num / den


def kernel(logits, labels):
    n, c = logits.shape

    rows_per = n // NSUB
    mesh = plsc.VectorSubcoreMesh(
        core_axis_name="c", subcore_axis_name="s", num_cores=1
    )
    sc_gather = pl.kernel(
        functools.partial(_sc_gather_body, rows_per=rows_per, ncols=c),
        out_type=jax.ShapeDtypeStruct((n,), jnp.float32),
        mesh=mesh,
        scratch_types=[
            pltpu.VMEM((rows_per,), jnp.int32),
            pltpu.VMEM((rows_per // IDX_CHUNK, IDX_CHUNK), jnp.int32),
            pltpu.VMEM((rows_per,), jnp.float32),
            pltpu.SemaphoreType.DMA,
        ],
    )
    x_lab = sc_gather(logits.reshape(-1), labels.astype(jnp.int32))

    block = 2048
    nblocks = n // block
    xl3 = x_lab.reshape(nblocks, block, 1)

    out = pl.pallas_call(
        functools.partial(_soft_ece_kernel, nblocks=nblocks),
        grid=(nblocks,),
        in_specs=[
            pl.BlockSpec((block, c), lambda i: (i, 0)),
            pl.BlockSpec((1, block, 1), lambda i: (i, 0, 0)),
        ],
        out_specs=pl.BlockSpec((1, 1), lambda i: (0, 0)),
        out_shape=jax.ShapeDtypeStruct((1, 1), jnp.float32),
        scratch_shapes=[pltpu.VMEM((3, PAD_BINS), jnp.float32)],
    )(logits, xl3)
    return out[0, 0]
